# Initial kernel scaffold; baseline (speedup 1.0000x reference)
import jax
import jax.numpy as jnp
from jax.experimental import pallas as pl


def kernel(x, edge_index, batch, image, W1, b1, W2, b2, Wu, bu, We, be, Wc, bc):
    raise NotImplementedError("write your pallas kernel here")


# speedup vs baseline: 19.6886x; 19.6886x over previous
import functools
import jax
import jax.numpy as jnp
from jax import lax
from jax.experimental import pallas as pl
from jax.experimental.pallas import tpu as pltpu
from jax.experimental.pallas import tpu_sc as plsc

N0 = 10000
E0 = 320000
BG = 100
H = 128
DV = 600
NCLS = 10

NP1, NP2, NP3 = 10240, 5120, 2560
NW = 32
EW = E0 // NW
CH = 80
NCHUNK = EW // CH
VPC = CH // 16


def _mesh():
    return plsc.VectorSubcoreMesh(core_axis_name="c", subcore_axis_name="s")


def _make_sc_cnt():
    @functools.partial(
        pl.kernel, mesh=_mesh(),
        out_type=[jax.ShapeDtypeStruct((2 * NP1,), jnp.float32),
                  jax.ShapeDtypeStruct((2 * NP2,), jnp.float32),
                  jax.ShapeDtypeStruct((2 * NP3,), jnp.float32)],
        scratch_types=[pltpu.VMEM((CH,), jnp.int32),
                       pltpu.VMEM((CH,), jnp.int32),
                       pltpu.VMEM((CH,), jnp.float32),
                       pltpu.VMEM((NP1 // 16,), jnp.float32),
                       pltpu.VMEM_SHARED((NP1,), jnp.float32),
                       pltpu.VMEM_SHARED((NP2,), jnp.float32),
                       pltpu.VMEM_SHARED((NP3,), jnp.float32)],
    )
    def k(dst_hbm, o1, o2, o3, d_v, i_v, ones_v, z_v, a1, a2, a3):
        cid = lax.axis_index("c")
        sid = lax.axis_index("s")
        w = cid * 16 + sid
        zer = jnp.zeros((16,), jnp.float32)
        one = jnp.ones((16,), jnp.float32)

        def zfill(j, _):
            z_v[pl.ds(j * 16, 16)] = zer
            return 0
        lax.fori_loop(0, NP1 // 256, zfill, 0)

        def ofill(j, _):
            ones_v[pl.ds(j * 16, 16)] = one
            return 0
        lax.fori_loop(0, VPC, ofill, 0)

        pltpu.sync_copy(z_v, a1.at[pl.ds(sid * (NP1 // 16), NP1 // 16)])
        pltpu.sync_copy(z_v.at[pl.ds(0, NP2 // 16)],
                        a2.at[pl.ds(sid * (NP2 // 16), NP2 // 16)])
        pltpu.sync_copy(z_v.at[pl.ds(0, NP3 // 16)],
                        a3.at[pl.ds(sid * (NP3 // 16), NP3 // 16)])
        plsc.subcore_barrier()

        def chunk(c, _):
            base = w * EW + c * CH
            pltpu.sync_copy(dst_hbm.at[pl.ds(base, CH)], d_v)
            pltpu.sync_copy(ones_v, a1.at[d_v], add=True)

            def sh1(j, _):
                i_v[pl.ds(j * 16, 16)] = lax.shift_right_logical(
                    d_v[pl.ds(j * 16, 16)], 1)
                return 0
            lax.fori_loop(0, VPC, sh1, 0)
            pltpu.sync_copy(ones_v, a2.at[i_v], add=True)

            def sh2(j, _):
                i_v[pl.ds(j * 16, 16)] = lax.shift_right_logical(
                    i_v[pl.ds(j * 16, 16)], 1)
                return 0
            lax.fori_loop(0, VPC, sh2, 0)
            pltpu.sync_copy(ones_v, a3.at[i_v], add=True)
            return 0
        lax.fori_loop(0, NCHUNK, chunk, 0)
        plsc.subcore_barrier()

        pltpu.sync_copy(a1.at[pl.ds(sid * (NP1 // 16), NP1 // 16)], z_v)
        pltpu.sync_copy(z_v,
                        o1.at[pl.ds(cid * NP1 + sid * (NP1 // 16), NP1 // 16)])
        pltpu.sync_copy(a2.at[pl.ds(sid * (NP2 // 16), NP2 // 16)],
                        z_v.at[pl.ds(0, NP2 // 16)])
        pltpu.sync_copy(z_v.at[pl.ds(0, NP2 // 16)],
                        o2.at[pl.ds(cid * NP2 + sid * (NP2 // 16), NP2 // 16)])
        pltpu.sync_copy(a3.at[pl.ds(sid * (NP3 // 16), NP3 // 16)],
                        z_v.at[pl.ds(0, NP3 // 16)])
        pltpu.sync_copy(z_v.at[pl.ds(0, NP3 // 16)],
                        o3.at[pl.ds(cid * NP3 + sid * (NP3 // 16), NP3 // 16)])
    return k


def _make_sc_msg(shift, np_pad, ch):
    share = np_pad // 16
    nchunk = EW // ch

    @functools.partial(
        pl.kernel, mesh=_mesh(),
        out_type=jax.ShapeDtypeStruct((2 * np_pad, H), jnp.float32),
        scratch_types=[pltpu.VMEM((ch,), jnp.int32),
                       pltpu.VMEM((ch,), jnp.int32),
                       pltpu.VMEM((ch, H), jnp.float32),
                       pltpu.VMEM((16, H), jnp.float32),
                       pltpu.VMEM_SHARED((np_pad, H), jnp.float32),
                       pltpu.SemaphoreType.DMA],
    )
    def k(y_hbm, src_hbm, dst_hbm, out, s_v, d_v, rows_v, zrow, s_sh, sem):
        cid = lax.axis_index("c")
        sid = lax.axis_index("s")
        w = cid * 16 + sid
        zer = jnp.zeros((16,), jnp.float32)

        def zfill(i, _):
            zrow[i // 8, pl.ds((i % 8) * 16, 16)] = zer
            return 0
        lax.fori_loop(0, 128, zfill, 0)

        def zcopy(kk, _):
            pltpu.sync_copy(zrow, s_sh.at[pl.ds(sid * share + kk * 16, 16), :])
            return 0
        lax.fori_loop(0, share // 16, zcopy, 0)
        plsc.subcore_barrier()

        def chunk(c, _):
            base = w * EW + c * ch
            pltpu.sync_copy(src_hbm.at[pl.ds(base, ch)], s_v)
            pltpu.sync_copy(dst_hbm.at[pl.ds(base, ch)], d_v)
            if shift:
                def sh(j, _):
                    s_v[pl.ds(j * 16, 16)] = lax.shift_right_logical(
                        s_v[pl.ds(j * 16, 16)], shift)
                    d_v[pl.ds(j * 16, 16)] = lax.shift_right_logical(
                        d_v[pl.ds(j * 16, 16)], shift)
                    return 0
                lax.fori_loop(0, ch // 16, sh, 0)
            pltpu.async_copy(y_hbm.at[s_v], rows_v, sem).wait()
            pltpu.sync_copy(rows_v, s_sh.at[d_v], add=True)
            return 0
        lax.fori_loop(0, nchunk, chunk, 0)
        plsc.subcore_barrier()

        pltpu.sync_copy(
            s_sh.at[pl.ds(sid * share, share), :],
            out.at[pl.ds(cid * np_pad + sid * share, share), :])
    return k


def _make_sc_esc(shift, np_pad):
    share = np_pad // 16

    @functools.partial(
        pl.kernel, mesh=_mesh(),
        out_type=jax.ShapeDtypeStruct((2 * np_pad,), jnp.float32),
        compiler_params=pltpu.CompilerParams(needs_layout_passes=False),
        scratch_types=[pltpu.VMEM((CH,), jnp.int32),
                       pltpu.VMEM((CH,), jnp.int32),
                       pltpu.VMEM((CH,), jnp.float32),
                       pltpu.VMEM((np_pad,), jnp.float32),
                       pltpu.VMEM((np_pad,), jnp.float32),
                       pltpu.VMEM((share,), jnp.float32),
                       pltpu.VMEM_SHARED((np_pad,), jnp.float32)],
    )
    def k(a_hbm, t_hbm, src_hbm, dst_hbm, out, s_v, d_v, vals_v,
          a_v, t_v, z_v, acc):
        cid = lax.axis_index("c")
        sid = lax.axis_index("s")
        w = cid * 16 + sid
        zer = jnp.zeros((16,), jnp.float32)
        pltpu.sync_copy(a_hbm, a_v)
        pltpu.sync_copy(t_hbm, t_v)

        def zfill(j, _):
            z_v[pl.ds(j * 16, 16)] = zer
            return 0
        lax.fori_loop(0, share // 16, zfill, 0)
        pltpu.sync_copy(z_v, acc.at[pl.ds(sid * share, share)])
        plsc.subcore_barrier()

        def chunk(c, _):
            base = w * EW + c * CH
            pltpu.sync_copy(src_hbm.at[pl.ds(base, CH)], s_v)
            pltpu.sync_copy(dst_hbm.at[pl.ds(base, CH)], d_v)

            def vreg(j, _):
                sv = s_v[pl.ds(j * 16, 16)]
                dv = d_v[pl.ds(j * 16, 16)]
                if shift:
                    sv = lax.shift_right_logical(sv, shift)
                    dv = lax.shift_right_logical(dv, shift)
                    d_v[pl.ds(j * 16, 16)] = dv
                av = plsc.load_gather(a_v, [sv])
                tv = plsc.load_gather(t_v, [dv])
                z = av + tv
                vals_v[pl.ds(j * 16, 16)] = 1.0 / (1.0 + jnp.exp(-z))
                return 0
            lax.fori_loop(0, VPC, vreg, 0)
            pltpu.sync_copy(vals_v, acc.at[d_v], add=True)
            return 0
        lax.fori_loop(0, NCHUNK, chunk, 0)
        plsc.subcore_barrier()

        pltpu.sync_copy(acc.at[pl.ds(sid * share, share)], z_v)
        pltpu.sync_copy(z_v, out.at[pl.ds(cid * np_pad + sid * share, share)])
    return k


def _tc_deg_body(c1a, c1b, c2a, c2b, c3a, c3b,
                 d1, d2, d3, l1, l2, l3):
    for ca, cb, d, l in ((c1a, c1b, d1, l1), (c2a, c2b, d2, l2),
                         (c3a, c3b, d3, l3)):
        c = ca[...] + cb[...]
        d[...] = lax.rsqrt(c + 1.0)
        l[...] = jnp.maximum(c, 1.0)


def _tc_deg(cp1, cp2, cp3):
    outs = [jax.ShapeDtypeStruct((n, 1), jnp.float32)
            for n in (NP1, NP2, NP3) for _ in (0, 1)]
    outs = [jax.ShapeDtypeStruct((NP1, 1), jnp.float32),
            jax.ShapeDtypeStruct((NP2, 1), jnp.float32),
            jax.ShapeDtypeStruct((NP3, 1), jnp.float32),
            jax.ShapeDtypeStruct((NP1, 1), jnp.float32),
            jax.ShapeDtypeStruct((NP2, 1), jnp.float32),
            jax.ShapeDtypeStruct((NP3, 1), jnp.float32)]
    fn = pl.pallas_call(
        lambda c1a, c1b, c2a, c2b, c3a, c3b, d1, d2, d3, l1, l2, l3:
        _tc_deg_body(c1a, c1b, c2a, c2b, c3a, c3b, d1, d2, d3, l1, l2, l3),
        out_shape=outs)
    return fn(cp1[:NP1].reshape(NP1, 1), cp1[NP1:].reshape(NP1, 1),
              cp2[:NP2].reshape(NP2, 1), cp2[NP2:].reshape(NP2, 1),
              cp3[:NP3].reshape(NP3, 1), cp3[NP3:].reshape(NP3, 1))


def _tc_img_body(img_ref, wd_ref, o1, o2, o3):
    x28 = img_ref[0]

    def hist_dvu(xim, n):
        idx = jnp.clip(jnp.floor(xim * DV).astype(jnp.int32), 0, DV - 1)
        bi = lax.broadcasted_iota(jnp.int32, (n, DV), 1)
        m = jnp.zeros((n, DV), jnp.float32)
        for q in range(n):
            m = m + (idx[:, q:q + 1] == bi).astype(jnp.float32)
        hrow = jnp.dot(jnp.ones((1, n), jnp.float32), m,
                       preferred_element_type=jnp.float32)
        return jnp.dot(hrow, wd_ref[...], preferred_element_type=jnp.float32)

    def pmat(n):
        r = lax.broadcasted_iota(jnp.int32, (n // 2, n), 0)
        c = lax.broadcasted_iota(jnp.int32, (n // 2, n), 1)
        return jnp.where((c == 2 * r) | (c == 2 * r + 1), 0.5, 0.0)

    hp = jax.lax.Precision.HIGHEST
    o1[...] = hist_dvu(x28, 28).reshape(1, 1, H)
    p14 = pmat(28)
    x14 = jnp.dot(jnp.dot(p14, x28, preferred_element_type=jnp.float32,
                          precision=hp),
                  p14.T, preferred_element_type=jnp.float32, precision=hp)
    o2[...] = hist_dvu(x14, 14).reshape(1, 1, H)
    p7 = pmat(14)
    x7 = jnp.dot(jnp.dot(p7, x14, preferred_element_type=jnp.float32,
                         precision=hp),
                 p7.T, preferred_element_type=jnp.float32, precision=hp)
    o3[...] = hist_dvu(x7, 7).reshape(1, 1, H)


def _tc_images(image, wu_d):
    outs = [jax.ShapeDtypeStruct((BG, 1, H), jnp.float32)] * 3
    fn = pl.pallas_call(
        _tc_img_body,
        grid=(BG,),
        in_specs=[pl.BlockSpec((1, 28, 28), lambda b: (b, 0, 0)),
                  pl.BlockSpec((DV, H), lambda b: (0, 0))],
        out_specs=[pl.BlockSpec((1, 1, H), lambda b: (b, 0, 0))] * 3,
        out_shape=outs)
    o1, o2, o3 = fn(image.reshape(BG, 28, 28), wu_d)
    return o1.reshape(BG, H), o2.reshape(BG, H), o3.reshape(BG, H)


def _tc_in_body(x_ref, w_ref, dinv_ref, y_ref):
    xw = jnp.dot(x_ref[...], w_ref[...], preferred_element_type=jnp.float32)
    y_ref[...] = xw * dinv_ref[...]


def _tc_in(x_pad, w, dinv):
    n = x_pad.shape[0]
    r = 2560
    fn = pl.pallas_call(
        _tc_in_body,
        grid=(n // r,),
        in_specs=[pl.BlockSpec((r, H), lambda i: (i, 0)),
                  pl.BlockSpec((H, H), lambda i: (0, 0)),
                  pl.BlockSpec((r, 1), lambda i: (i, 0))],
        out_specs=pl.BlockSpec((r, H), lambda i: (i, 0)),
        out_shape=jax.ShapeDtypeStruct((n, H), jnp.float32))
    return fn(x_pad, w, dinv)


def _tc_a_body(s0, s1, y, dinv, bvec, wux, dvu, bat, buv, wea, web, bev,
               h_ref, a_ref, t_ref):
    g = jnp.maximum(dinv[...] * (s0[...] + s1[...] + y[...]) + bvec[...], 0.0)
    bi = lax.broadcasted_iota(jnp.int32, (g.shape[0], BG), 1)
    oh = (bat[...] == bi).astype(jnp.float32)
    h = jnp.dot(g, wux[...], preferred_element_type=jnp.float32)
    h = h + jnp.dot(oh, dvu[...], preferred_element_type=jnp.float32)
    h = jnp.maximum(h + buv[...], 0.0)
    h_ref[...] = h
    a_ref[...] = jnp.dot(h, wea[...],
                         preferred_element_type=jnp.float32) + bev[...]
    t_ref[...] = jnp.dot(h, web[...], preferred_element_type=jnp.float32)


def _tc_a(np_pad, s0, s1, y, dinv, bvec, wux, dvu, bat, buv, wea, web, bev):
    r = 2560
    outs = [jax.ShapeDtypeStruct((np_pad, H), jnp.float32),
            jax.ShapeDtypeStruct((np_pad, 1), jnp.float32),
            jax.ShapeDtypeStruct((np_pad, 1), jnp.float32)]
    fn = pl.pallas_call(
        _tc_a_body,
        grid=(np_pad // r,),
        in_specs=[pl.BlockSpec((r, H), lambda i: (i, 0)),
                  pl.BlockSpec((r, H), lambda i: (i, 0)),
                  pl.BlockSpec((r, H), lambda i: (i, 0)),
                  pl.BlockSpec((r, 1), lambda i: (i, 0)),
                  pl.BlockSpec((1, H), lambda i: (0, 0)),
                  pl.BlockSpec((H, H), lambda i: (0, 0)),
                  pl.BlockSpec((BG, H), lambda i: (0, 0)),
                  pl.BlockSpec((r, 1), lambda i: (i, 0)),
                  pl.BlockSpec((1, H), lambda i: (0, 0)),
                  pl.BlockSpec((H, 1), lambda i: (0, 0)),
                  pl.BlockSpec((H, 1), lambda i: (0, 0)),
                  pl.BlockSpec((1, 1), lambda i: (0, 0))],
        out_specs=[pl.BlockSpec((r, H), lambda i: (i, 0)),
                   pl.BlockSpec((r, 1), lambda i: (i, 0)),
                   pl.BlockSpec((r, 1), lambda i: (i, 0))],
        out_shape=outs)
    return fn(s0, s1, y, dinv, bvec, wux, dvu, bat, buv, wea, web, bev)


def _tc_b_body(na, nb, cl, h, w2, dinv2, y2_ref):
    nsc = (na[...] + nb[...]) / cl[...]
    xg = h[...] * nsc
    r2 = y2_ref.shape[0]
    x2 = xg.reshape(r2, 2, H).sum(axis=1) * 0.5
    xw = jnp.dot(x2, w2[...], preferred_element_type=jnp.float32)
    y2_ref[...] = xw * dinv2[...]


def _tc_b(np_pad, na, nb, cl, h, w2, dinv2):
    r = 2560
    n2 = np_pad // 2
    fn = pl.pallas_call(
        _tc_b_body,
        grid=(n2 // r,),
        in_specs=[pl.BlockSpec((2 * r, 1), lambda i: (i, 0)),
                  pl.BlockSpec((2 * r, 1), lambda i: (i, 0)),
                  pl.BlockSpec((2 * r, 1), lambda i: (i, 0)),
                  pl.BlockSpec((2 * r, H), lambda i: (i, 0)),
                  pl.BlockSpec((H, H), lambda i: (0, 0)),
                  pl.BlockSpec((r, 1), lambda i: (i, 0))],
        out_specs=pl.BlockSpec((r, H), lambda i: (i, 0)),
        out_shape=jax.ShapeDtypeStruct((n2, H), jnp.float32))
    return fn(na, nb, cl, h, w2, dinv2)


def _tc_c_body(na, nb, cl, h, bat4, wc, bcv, out_ref):
    nsc = (na[...] + nb[...]) / cl[...]
    xg = h[...] * nsc
    x4 = xg.reshape(NP3 // 2, 2, H).sum(axis=1) * 0.5
    bi = lax.broadcasted_iota(jnp.int32, (NP3 // 2, BG), 1)
    ri = lax.broadcasted_iota(jnp.int32, (NP3 // 2, BG), 0)
    oh = ((bat4[...] == bi) & (ri < N0 // 8)).astype(jnp.float32)
    dn = (((0,), (0,)), ((), ()))
    gsum = lax.dot_general(oh, x4, dn, preferred_element_type=jnp.float32)
    gcnt = lax.dot_general(oh, jnp.ones((NP3 // 2, 1), jnp.float32), dn,
                           preferred_element_type=jnp.float32)
    gx = gsum / jnp.maximum(gcnt, 1.0)
    out_ref[...] = jnp.dot(gx, wc[...],
                           preferred_element_type=jnp.float32) + bcv[...]


def _tc_c(na, nb, cl, h, bat4, wc, bcv):
    fn = pl.pallas_call(
        _tc_c_body,
        out_shape=jax.ShapeDtypeStruct((BG, NCLS), jnp.float32))
    return fn(na, nb, cl, h, bat4, wc, bcv)


_NPS = (NP1, NP2, NP3)


def kernel(x, edge_index, batch, image, W1, b1, W2, b2, Wu, bu, We, be,
           Wc, bc):
    src = edge_index[0]
    dst = edge_index[1]
    sc_cnt = _make_sc_cnt()
    cp1, cp2, cp3 = sc_cnt(dst)
    d1, d2, d3, l1, l2, l3 = _tc_deg(cp1, cp2, cp3)
    dinvs = (d1, d2, d3)
    clips = (l1, l2, l3)

    dvu1, dvu2, dvu3 = _tc_images(image, Wu[H:])
    dvus = (dvu1, dvu2, dvu3)

    x_pad = jnp.pad(x, ((0, NP1 - N0), (0, 0)))
    y = _tc_in(x_pad, W1, d1)

    wu_x = Wu[:H]
    wea = We[:H]
    web = We[H:]
    bev = be.reshape(1, 1)
    buv = bu.reshape(1, H)
    bvecs = (b1.reshape(1, H), b2.reshape(1, H), b2.reshape(1, H))

    out = None
    for s in range(3):
        np_pad = _NPS[s]
        n = N0 >> s
        sp = _make_sc_msg(s, np_pad, CH)(y, src, dst)
        bat = jnp.pad(batch[::1 << s], (0, np_pad - n)).reshape(np_pad, 1)
        h, a, t = _tc_a(np_pad, sp[:np_pad], sp[np_pad:], y, dinvs[s],
                        bvecs[s], wu_x, dvus[s], bat, buv, wea, web, bev)
        np_part = _make_sc_esc(s, np_pad)(
            a.reshape(np_pad), t.reshape(np_pad), src, dst)
        na = np_part[:np_pad].reshape(np_pad, 1)
        nb = np_part[np_pad:].reshape(np_pad, 1)
        if s < 2:
            y = _tc_b(np_pad, na, nb, clips[s], h, W2, dinvs[s + 1])
        else:
            bat4 = jnp.pad(batch[::8], (0, NP3 // 2 - N0 // 8)).reshape(
                NP3 // 2, 1)
            out = _tc_c(na, nb, clips[2], h, bat4, Wc, bc.reshape(1, NCLS))
    return out


# speedup vs baseline: 25.7890x; 1.3098x over previous
import functools
import jax
import jax.numpy as jnp
from jax import lax
from jax.experimental import pallas as pl
from jax.experimental.pallas import tpu as pltpu
from jax.experimental.pallas import tpu_sc as plsc

N0 = 10000
E0 = 320000
BG = 100
H = 128
DV = 600
NCLS = 10

NP1, NP2, NP3 = 10240, 5120, 2560
NW = 32
EW = E0 // NW
CH = 80
NCHUNK = EW // CH
VPC = CH // 16


def _mesh():
    return plsc.VectorSubcoreMesh(core_axis_name="c", subcore_axis_name="s")


def _make_sc_cnt():
    @functools.partial(
        pl.kernel, mesh=_mesh(),
        out_type=[jax.ShapeDtypeStruct((2 * NP1,), jnp.float32),
                  jax.ShapeDtypeStruct((2 * NP2,), jnp.float32),
                  jax.ShapeDtypeStruct((2 * NP3,), jnp.float32)],
        scratch_types=[pltpu.VMEM((CH,), jnp.int32),
                       pltpu.VMEM((CH,), jnp.int32),
                       pltpu.VMEM((CH,), jnp.float32),
                       pltpu.VMEM((NP1 // 16,), jnp.float32),
                       pltpu.VMEM_SHARED((NP1,), jnp.float32),
                       pltpu.VMEM_SHARED((NP2,), jnp.float32),
                       pltpu.VMEM_SHARED((NP3,), jnp.float32)],
    )
    def k(dst_hbm, o1, o2, o3, d_v, i_v, ones_v, z_v, a1, a2, a3):
        cid = lax.axis_index("c")
        sid = lax.axis_index("s")
        w = cid * 16 + sid
        zer = jnp.zeros((16,), jnp.float32)
        one = jnp.ones((16,), jnp.float32)

        def zfill(j, _):
            z_v[pl.ds(j * 16, 16)] = zer
            return 0
        lax.fori_loop(0, NP1 // 256, zfill, 0)

        def ofill(j, _):
            ones_v[pl.ds(j * 16, 16)] = one
            return 0
        lax.fori_loop(0, VPC, ofill, 0)

        pltpu.sync_copy(z_v, a1.at[pl.ds(sid * (NP1 // 16), NP1 // 16)])
        pltpu.sync_copy(z_v.at[pl.ds(0, NP2 // 16)],
                        a2.at[pl.ds(sid * (NP2 // 16), NP2 // 16)])
        pltpu.sync_copy(z_v.at[pl.ds(0, NP3 // 16)],
                        a3.at[pl.ds(sid * (NP3 // 16), NP3 // 16)])
        plsc.subcore_barrier()

        def chunk(c, _):
            base = w * EW + c * CH
            pltpu.sync_copy(dst_hbm.at[pl.ds(base, CH)], d_v)
            pltpu.sync_copy(ones_v, a1.at[d_v], add=True)

            def sh1(j, _):
                i_v[pl.ds(j * 16, 16)] = lax.shift_right_logical(
                    d_v[pl.ds(j * 16, 16)], 1)
                return 0
            lax.fori_loop(0, VPC, sh1, 0)
            pltpu.sync_copy(ones_v, a2.at[i_v], add=True)

            def sh2(j, _):
                i_v[pl.ds(j * 16, 16)] = lax.shift_right_logical(
                    i_v[pl.ds(j * 16, 16)], 1)
                return 0
            lax.fori_loop(0, VPC, sh2, 0)
            pltpu.sync_copy(ones_v, a3.at[i_v], add=True)
            return 0
        lax.fori_loop(0, NCHUNK, chunk, 0)
        plsc.subcore_barrier()

        pltpu.sync_copy(a1.at[pl.ds(sid * (NP1 // 16), NP1 // 16)], z_v)
        pltpu.sync_copy(z_v,
                        o1.at[pl.ds(cid * NP1 + sid * (NP1 // 16), NP1 // 16)])
        pltpu.sync_copy(a2.at[pl.ds(sid * (NP2 // 16), NP2 // 16)],
                        z_v.at[pl.ds(0, NP2 // 16)])
        pltpu.sync_copy(z_v.at[pl.ds(0, NP2 // 16)],
                        o2.at[pl.ds(cid * NP2 + sid * (NP2 // 16), NP2 // 16)])
        pltpu.sync_copy(a3.at[pl.ds(sid * (NP3 // 16), NP3 // 16)],
                        z_v.at[pl.ds(0, NP3 // 16)])
        pltpu.sync_copy(z_v.at[pl.ds(0, NP3 // 16)],
                        o3.at[pl.ds(cid * NP3 + sid * (NP3 // 16), NP3 // 16)])
    return k


def _make_sc_msg(shift, np_pad, ch):
    share = np_pad // 16
    nchunk = EW // ch

    @functools.partial(
        pl.kernel, mesh=_mesh(),
        out_type=jax.ShapeDtypeStruct((2 * np_pad, H), jnp.float32),
        scratch_types=[pltpu.VMEM((ch,), jnp.int32),
                       pltpu.VMEM((ch,), jnp.int32),
                       pltpu.VMEM((ch,), jnp.int32),
                       pltpu.VMEM((ch,), jnp.int32),
                       pltpu.VMEM((ch, H), jnp.float32),
                       pltpu.VMEM((ch, H), jnp.float32),
                       pltpu.VMEM((16, H), jnp.float32),
                       pltpu.VMEM_SHARED((np_pad, H), jnp.float32),
                       pltpu.SemaphoreType.DMA,
                       pltpu.SemaphoreType.DMA],
    )
    def k(y_hbm, src_hbm, dst_hbm, out, s_v0, d_v0, s_v1, d_v1,
          rows0, rows1, zrow, s_sh, sem0, sem1):
        cid = lax.axis_index("c")
        sid = lax.axis_index("s")
        w = cid * 16 + sid
        zer = jnp.zeros((16,), jnp.float32)

        def zfill(i, _):
            zrow[i // 8, pl.ds((i % 8) * 16, 16)] = zer
            return 0
        lax.fori_loop(0, 128, zfill, 0)

        def zcopy(kk, _):
            pltpu.sync_copy(zrow, s_sh.at[pl.ds(sid * share + kk * 16, 16), :])
            return 0
        lax.fori_loop(0, share // 16, zcopy, 0)
        plsc.subcore_barrier()

        def prep(c, s_v, d_v):
            base = w * EW + c * ch
            pltpu.sync_copy(src_hbm.at[pl.ds(base, ch)], s_v)
            pltpu.sync_copy(dst_hbm.at[pl.ds(base, ch)], d_v)
            if shift:
                def sh(j, _):
                    s_v[pl.ds(j * 16, 16)] = lax.shift_right_logical(
                        s_v[pl.ds(j * 16, 16)], shift)
                    d_v[pl.ds(j * 16, 16)] = lax.shift_right_logical(
                        d_v[pl.ds(j * 16, 16)], shift)
                    return 0
                lax.fori_loop(0, ch // 16, sh, 0)

        prep(0, s_v0, d_v0)
        pltpu.async_copy(y_hbm.at[s_v0], rows0, sem0)

        def pair(cc, _):
            c1 = 2 * cc + 1
            prep(c1, s_v1, d_v1)
            pltpu.async_copy(y_hbm.at[s_v1], rows1, sem1)
            pltpu.make_async_copy(y_hbm.at[s_v0], rows0, sem0).wait()
            pltpu.sync_copy(rows0, s_sh.at[d_v0], add=True)
            prep(c1 + 1, s_v0, d_v0)
            pltpu.async_copy(y_hbm.at[s_v0], rows0, sem0)
            pltpu.make_async_copy(y_hbm.at[s_v1], rows1, sem1).wait()
            pltpu.sync_copy(rows1, s_sh.at[d_v1], add=True)
            return 0
        lax.fori_loop(0, nchunk // 2, pair, 0)
        pltpu.make_async_copy(y_hbm.at[s_v0], rows0, sem0).wait()
        pltpu.sync_copy(rows0, s_sh.at[d_v0], add=True)
        plsc.subcore_barrier()

        pltpu.sync_copy(
            s_sh.at[pl.ds(sid * share, share), :],
            out.at[pl.ds(cid * np_pad + sid * share, share), :])
    return k


def _make_sc_esc(shift, np_pad):
    share = np_pad // 16

    @functools.partial(
        pl.kernel, mesh=_mesh(),
        out_type=jax.ShapeDtypeStruct((2 * np_pad,), jnp.float32),
        compiler_params=pltpu.CompilerParams(needs_layout_passes=False),
        scratch_types=[pltpu.VMEM((CH,), jnp.int32),
                       pltpu.VMEM((CH,), jnp.int32),
                       pltpu.VMEM((CH,), jnp.float32),
                       pltpu.VMEM((np_pad,), jnp.float32),
                       pltpu.VMEM((np_pad,), jnp.float32),
                       pltpu.VMEM((share,), jnp.float32),
                       pltpu.VMEM_SHARED((np_pad,), jnp.float32)],
    )
    def k(a_hbm, t_hbm, src_hbm, dst_hbm, out, s_v, d_v, vals_v,
          a_v, t_v, z_v, acc):
        cid = lax.axis_index("c")
        sid = lax.axis_index("s")
        w = cid * 16 + sid
        zer = jnp.zeros((16,), jnp.float32)
        pltpu.sync_copy(a_hbm, a_v)
        pltpu.sync_copy(t_hbm, t_v)

        def zfill(j, _):
            z_v[pl.ds(j * 16, 16)] = zer
            return 0
        lax.fori_loop(0, share // 16, zfill, 0)
        pltpu.sync_copy(z_v, acc.at[pl.ds(sid * share, share)])
        plsc.subcore_barrier()

        def chunk(c, _):
            base = w * EW + c * CH
            pltpu.sync_copy(src_hbm.at[pl.ds(base, CH)], s_v)
            pltpu.sync_copy(dst_hbm.at[pl.ds(base, CH)], d_v)

            def vreg(j, _):
                sv = s_v[pl.ds(j * 16, 16)]
                dv = d_v[pl.ds(j * 16, 16)]
                if shift:
                    sv = lax.shift_right_logical(sv, shift)
                    dv = lax.shift_right_logical(dv, shift)
                    d_v[pl.ds(j * 16, 16)] = dv
                av = plsc.load_gather(a_v, [sv])
                tv = plsc.load_gather(t_v, [dv])
                z = av + tv
                vals_v[pl.ds(j * 16, 16)] = 1.0 / (1.0 + jnp.exp(-z))
                return 0
            lax.fori_loop(0, VPC, vreg, 0)
            pltpu.sync_copy(vals_v, acc.at[d_v], add=True)
            return 0
        lax.fori_loop(0, NCHUNK, chunk, 0)
        plsc.subcore_barrier()

        pltpu.sync_copy(acc.at[pl.ds(sid * share, share)], z_v)
        pltpu.sync_copy(z_v, out.at[pl.ds(cid * np_pad + sid * share, share)])
    return k


def _tc_deg_body(c1a, c1b, c2a, c2b, c3a, c3b,
                 d1, d2, d3, l1, l2, l3):
    for ca, cb, d, l in ((c1a, c1b, d1, l1), (c2a, c2b, d2, l2),
                         (c3a, c3b, d3, l3)):
        c = ca[...] + cb[...]
        d[...] = lax.rsqrt(c + 1.0)
        l[...] = jnp.maximum(c, 1.0)


def _tc_deg(cp1, cp2, cp3):
    outs = [jax.ShapeDtypeStruct((n, 1), jnp.float32)
            for n in (NP1, NP2, NP3) for _ in (0, 1)]
    outs = [jax.ShapeDtypeStruct((NP1, 1), jnp.float32),
            jax.ShapeDtypeStruct((NP2, 1), jnp.float32),
            jax.ShapeDtypeStruct((NP3, 1), jnp.float32),
            jax.ShapeDtypeStruct((NP1, 1), jnp.float32),
            jax.ShapeDtypeStruct((NP2, 1), jnp.float32),
            jax.ShapeDtypeStruct((NP3, 1), jnp.float32)]
    fn = pl.pallas_call(
        lambda c1a, c1b, c2a, c2b, c3a, c3b, d1, d2, d3, l1, l2, l3:
        _tc_deg_body(c1a, c1b, c2a, c2b, c3a, c3b, d1, d2, d3, l1, l2, l3),
        out_shape=outs)
    return fn(cp1[:NP1].reshape(NP1, 1), cp1[NP1:].reshape(NP1, 1),
              cp2[:NP2].reshape(NP2, 1), cp2[NP2:].reshape(NP2, 1),
              cp3[:NP3].reshape(NP3, 1), cp3[NP3:].reshape(NP3, 1))


def _tc_img_body(img_ref, wd_ref, o1, o2, o3):
    x28 = img_ref[0]

    def hist_dvu(xim, n):
        idx = jnp.clip(jnp.floor(xim * DV).astype(jnp.int32), 0, DV - 1)
        bi = lax.broadcasted_iota(jnp.int32, (n, DV), 1)
        m = jnp.zeros((n, DV), jnp.float32)
        for q in range(n):
            m = m + (idx[:, q:q + 1] == bi).astype(jnp.float32)
        hrow = jnp.dot(jnp.ones((1, n), jnp.float32), m,
                       preferred_element_type=jnp.float32)
        return jnp.dot(hrow, wd_ref[...], preferred_element_type=jnp.float32)

    def pmat(n):
        r = lax.broadcasted_iota(jnp.int32, (n // 2, n), 0)
        c = lax.broadcasted_iota(jnp.int32, (n // 2, n), 1)
        return jnp.where((c == 2 * r) | (c == 2 * r + 1), 0.5, 0.0)

    hp = jax.lax.Precision.HIGHEST
    o1[...] = hist_dvu(x28, 28).reshape(1, 1, H)
    p14 = pmat(28)
    x14 = jnp.dot(jnp.dot(p14, x28, preferred_element_type=jnp.float32,
                          precision=hp),
                  p14.T, preferred_element_type=jnp.float32, precision=hp)
    o2[...] = hist_dvu(x14, 14).reshape(1, 1, H)
    p7 = pmat(14)
    x7 = jnp.dot(jnp.dot(p7, x14, preferred_element_type=jnp.float32,
                         precision=hp),
                 p7.T, preferred_element_type=jnp.float32, precision=hp)
    o3[...] = hist_dvu(x7, 7).reshape(1, 1, H)


def _tc_images(image, wu_d):
    outs = [jax.ShapeDtypeStruct((BG, 1, H), jnp.float32)] * 3
    fn = pl.pallas_call(
        _tc_img_body,
        grid=(BG,),
        in_specs=[pl.BlockSpec((1, 28, 28), lambda b: (b, 0, 0)),
                  pl.BlockSpec((DV, H), lambda b: (0, 0))],
        out_specs=[pl.BlockSpec((1, 1, H), lambda b: (b, 0, 0))] * 3,
        out_shape=outs)
    o1, o2, o3 = fn(image.reshape(BG, 28, 28), wu_d)
    return o1.reshape(BG, H), o2.reshape(BG, H), o3.reshape(BG, H)


def _tc_in_body(x_ref, w_ref, dinv_ref, y_ref):
    xw = jnp.dot(x_ref[...], w_ref[...], preferred_element_type=jnp.float32)
    y_ref[...] = xw * dinv_ref[...]


def _tc_in(x_pad, w, dinv):
    n = x_pad.shape[0]
    r = 2560
    fn = pl.pallas_call(
        _tc_in_body,
        grid=(n // r,),
        in_specs=[pl.BlockSpec((r, H), lambda i: (i, 0)),
                  pl.BlockSpec((H, H), lambda i: (0, 0)),
                  pl.BlockSpec((r, 1), lambda i: (i, 0))],
        out_specs=pl.BlockSpec((r, H), lambda i: (i, 0)),
        out_shape=jax.ShapeDtypeStruct((n, H), jnp.float32))
    return fn(x_pad, w, dinv)


def _tc_a_body(s0, s1, y, dinv, bvec, wux, dvu, bat, buv, wea, web, bev,
               h_ref, a_ref, t_ref):
    g = jnp.maximum(dinv[...] * (s0[...] + s1[...] + y[...]) + bvec[...], 0.0)
    bi = lax.broadcasted_iota(jnp.int32, (g.shape[0], BG), 1)
    oh = (bat[...] == bi).astype(jnp.float32)
    h = jnp.dot(g, wux[...], preferred_element_type=jnp.float32)
    h = h + jnp.dot(oh, dvu[...], preferred_element_type=jnp.float32)
    h = jnp.maximum(h + buv[...], 0.0)
    h_ref[...] = h
    a_ref[...] = jnp.dot(h, wea[...],
                         preferred_element_type=jnp.float32) + bev[...]
    t_ref[...] = jnp.dot(h, web[...], preferred_element_type=jnp.float32)


def _tc_a(np_pad, s0, s1, y, dinv, bvec, wux, dvu, bat, buv, wea, web, bev):
    r = 2560
    outs = [jax.ShapeDtypeStruct((np_pad, H), jnp.float32),
            jax.ShapeDtypeStruct((np_pad, 1), jnp.float32),
            jax.ShapeDtypeStruct((np_pad, 1), jnp.float32)]
    fn = pl.pallas_call(
        _tc_a_body,
        grid=(np_pad // r,),
        in_specs=[pl.BlockSpec((r, H), lambda i: (i, 0)),
                  pl.BlockSpec((r, H), lambda i: (i, 0)),
                  pl.BlockSpec((r, H), lambda i: (i, 0)),
                  pl.BlockSpec((r, 1), lambda i: (i, 0)),
                  pl.BlockSpec((1, H), lambda i: (0, 0)),
                  pl.BlockSpec((H, H), lambda i: (0, 0)),
                  pl.BlockSpec((BG, H), lambda i: (0, 0)),
                  pl.BlockSpec((r, 1), lambda i: (i, 0)),
                  pl.BlockSpec((1, H), lambda i: (0, 0)),
                  pl.BlockSpec((H, 1), lambda i: (0, 0)),
                  pl.BlockSpec((H, 1), lambda i: (0, 0)),
                  pl.BlockSpec((1, 1), lambda i: (0, 0))],
        out_specs=[pl.BlockSpec((r, H), lambda i: (i, 0)),
                   pl.BlockSpec((r, 1), lambda i: (i, 0)),
                   pl.BlockSpec((r, 1), lambda i: (i, 0))],
        out_shape=outs)
    return fn(s0, s1, y, dinv, bvec, wux, dvu, bat, buv, wea, web, bev)


def _tc_b_body(na, nb, cl, h, w2, dinv2, y2_ref):
    nsc = (na[...] + nb[...]) / cl[...]
    xg = h[...] * nsc
    r2 = y2_ref.shape[0]
    x2 = xg.reshape(r2, 2, H).sum(axis=1) * 0.5
    xw = jnp.dot(x2, w2[...], preferred_element_type=jnp.float32)
    y2_ref[...] = xw * dinv2[...]


def _tc_b(np_pad, na, nb, cl, h, w2, dinv2):
    r = 2560
    n2 = np_pad // 2
    fn = pl.pallas_call(
        _tc_b_body,
        grid=(n2 // r,),
        in_specs=[pl.BlockSpec((2 * r, 1), lambda i: (i, 0)),
                  pl.BlockSpec((2 * r, 1), lambda i: (i, 0)),
                  pl.BlockSpec((2 * r, 1), lambda i: (i, 0)),
                  pl.BlockSpec((2 * r, H), lambda i: (i, 0)),
                  pl.BlockSpec((H, H), lambda i: (0, 0)),
                  pl.BlockSpec((r, 1), lambda i: (i, 0))],
        out_specs=pl.BlockSpec((r, H), lambda i: (i, 0)),
        out_shape=jax.ShapeDtypeStruct((n2, H), jnp.float32))
    return fn(na, nb, cl, h, w2, dinv2)


def _tc_c_body(na, nb, cl, h, bat4, wc, bcv, out_ref):
    nsc = (na[...] + nb[...]) / cl[...]
    xg = h[...] * nsc
    x4 = xg.reshape(NP3 // 2, 2, H).sum(axis=1) * 0.5
    bi = lax.broadcasted_iota(jnp.int32, (NP3 // 2, BG), 1)
    ri = lax.broadcasted_iota(jnp.int32, (NP3 // 2, BG), 0)
    oh = ((bat4[...] == bi) & (ri < N0 // 8)).astype(jnp.float32)
    dn = (((0,), (0,)), ((), ()))
    gsum = lax.dot_general(oh, x4, dn, preferred_element_type=jnp.float32)
    gcnt = lax.dot_general(oh, jnp.ones((NP3 // 2, 1), jnp.float32), dn,
                           preferred_element_type=jnp.float32)
    gx = gsum / jnp.maximum(gcnt, 1.0)
    out_ref[...] = jnp.dot(gx, wc[...],
                           preferred_element_type=jnp.float32) + bcv[...]


def _tc_c(na, nb, cl, h, bat4, wc, bcv):
    fn = pl.pallas_call(
        _tc_c_body,
        out_shape=jax.ShapeDtypeStruct((BG, NCLS), jnp.float32))
    return fn(na, nb, cl, h, bat4, wc, bcv)


_NPS = (NP1, NP2, NP3)


def kernel(x, edge_index, batch, image, W1, b1, W2, b2, Wu, bu, We, be,
           Wc, bc):
    src = edge_index[0]
    dst = edge_index[1]
    sc_cnt = _make_sc_cnt()
    cp1, cp2, cp3 = sc_cnt(dst)
    d1, d2, d3, l1, l2, l3 = _tc_deg(cp1, cp2, cp3)
    dinvs = (d1, d2, d3)
    clips = (l1, l2, l3)

    dvu1, dvu2, dvu3 = _tc_images(image, Wu[H:])
    dvus = (dvu1, dvu2, dvu3)

    x_pad = jnp.pad(x, ((0, NP1 - N0), (0, 0)))
    y = _tc_in(x_pad, W1, d1)

    wu_x = Wu[:H]
    wea = We[:H]
    web = We[H:]
    bev = be.reshape(1, 1)
    buv = bu.reshape(1, H)
    bvecs = (b1.reshape(1, H), b2.reshape(1, H), b2.reshape(1, H))

    out = None
    for s in range(3):
        np_pad = _NPS[s]
        n = N0 >> s
        sp = _make_sc_msg(s, np_pad, CH)(y, src, dst)
        bat = jnp.pad(batch[::1 << s], (0, np_pad - n)).reshape(np_pad, 1)
        h, a, t = _tc_a(np_pad, sp[:np_pad], sp[np_pad:], y, dinvs[s],
                        bvecs[s], wu_x, dvus[s], bat, buv, wea, web, bev)
        np_part = _make_sc_esc(s, np_pad)(
            a.reshape(np_pad), t.reshape(np_pad), src, dst)
        na = np_part[:np_pad].reshape(np_pad, 1)
        nb = np_part[np_pad:].reshape(np_pad, 1)
        if s < 2:
            y = _tc_b(np_pad, na, nb, clips[s], h, W2, dinvs[s + 1])
        else:
            bat4 = jnp.pad(batch[::8], (0, NP3 // 2 - N0 // 8)).reshape(
                NP3 // 2, 1)
            out = _tc_c(na, nb, clips[2], h, bat4, Wc, bc.reshape(1, NCLS))
    return out
